# Initial kernel scaffold; baseline (speedup 1.0000x reference)
import functools

import jax
import jax.numpy as jnp
from jax import lax
from jax.experimental import pallas as pl
from jax.experimental.pallas import tpu as pltpu
from jax.experimental.pallas import tpu_sc as plsc

B = 4096
L = 200
D = 60
DP = 64
NC = 2
NS = 16
NW = NC * NS
RPW = B // NW
HALF = L // 2


def _pool_body(x2_hbm, tbl_hbm, out_hbm, idx_v, buf0, buf1, stage, sem0, sem1):
    wid = lax.axis_index("s") * NC + lax.axis_index("c")
    pltpu.sync_copy(x2_hbm.at[pl.ds(wid * 2 * RPW, 2 * RPW)], idx_v)

    def fire(row, buf, sem):
        pltpu.async_copy(tbl_hbm.at[idx_v.at[2 * row]], buf.at[pl.ds(0, HALF)], sem)
        pltpu.async_copy(tbl_hbm.at[idx_v.at[2 * row + 1]], buf.at[pl.ds(HALF, HALF)], sem)

    def drain(buf, sem):
        pltpu.make_async_copy(tbl_hbm.at[pl.ds(0, L)], buf, sem).wait()

    def accum(buf, row):
        zero = jnp.zeros((16,), jnp.float32)

        def body(i, accs):
            a0, a1, a2, a3 = accs
            for u in range(4):
                r = 4 * i + u
                a0 = a0 + buf[r, pl.ds(0, 16)]
                a1 = a1 + buf[r, pl.ds(16, 16)]
                a2 = a2 + buf[r, pl.ds(32, 16)]
                a3 = a3 + buf[r, pl.ds(44, 16)]
            return a0, a1, a2, a3

        a0, a1, a2, a3 = lax.fori_loop(0, L // 4, body, (zero, zero, zero, zero))
        stage[row, pl.ds(0, 16)] = a0
        stage[row, pl.ds(16, 16)] = a1
        stage[row, pl.ds(32, 16)] = a2
        stage[row, pl.ds(48, 16)] = a3

    fire(0, buf0, sem0)
    fire(1, buf1, sem1)

    def outer(g, carry):
        b0 = 2 * g
        drain(buf0, sem0)
        accum(buf0, b0)

        @pl.when(g < RPW // 2 - 1)
        def _():
            fire(b0 + 2, buf0, sem0)

        drain(buf1, sem1)
        accum(buf1, b0 + 1)

        @pl.when(g < RPW // 2 - 1)
        def _():
            fire(b0 + 3, buf1, sem1)

        return carry

    lax.fori_loop(0, RPW // 2, outer, 0)
    pltpu.sync_copy(stage, out_hbm.at[pl.ds(wid * RPW, RPW)])


_pool = functools.partial(
    pl.kernel,
    out_type=jax.ShapeDtypeStruct((B, DP), jnp.float32),
    mesh=plsc.VectorSubcoreMesh(core_axis_name="c", subcore_axis_name="s"),
    scratch_types=[
        pltpu.VMEM((2 * RPW, HALF), jnp.int32),
        pltpu.VMEM((L, D), jnp.float32),
        pltpu.VMEM((L, D), jnp.float32),
        pltpu.VMEM((RPW, DP), jnp.float32),
        pltpu.SemaphoreType.DMA,
        pltpu.SemaphoreType.DMA,
    ],
)(_pool_body)


def _mlp_body(p_ref, il_ref, w2_ref, b2_ref, w1_ref, b1_ref, o_ref):
    x = p_ref[...] / il_ref[...]
    h = jnp.dot(x, w2_ref[...], preferred_element_type=jnp.float32) + b2_ref[...]
    h = jnp.maximum(h, 0.0)
    o_ref[...] = jnp.dot(h, w1_ref[...], preferred_element_type=jnp.float32) + b1_ref[...]


BT = 512


def _mlp(pooled, lenf, w2p, b2r, w1t, b1r):
    return pl.pallas_call(
        _mlp_body,
        grid=(B // BT,),
        in_specs=[
            pl.BlockSpec((BT, DP), lambda i: (i, 0)),
            pl.BlockSpec((BT, 1), lambda i: (i, 0)),
            pl.BlockSpec((DP, D), lambda i: (0, 0)),
            pl.BlockSpec((1, D), lambda i: (0, 0)),
            pl.BlockSpec((D, D), lambda i: (0, 0)),
            pl.BlockSpec((1, D), lambda i: (0, 0)),
        ],
        out_specs=pl.BlockSpec((BT, D), lambda i: (i, 0)),
        out_shape=jax.ShapeDtypeStruct((B, D), jnp.float32),
    )(pooled, lenf, w2p, b2r, w1t, b1r)


def kernel(x, lengths, table, W2, b2, W1, b1):
    x2 = x.astype(jnp.int32).reshape(2 * B, HALF)
    pooled = _pool(x2, table)
    lenf = lengths.astype(jnp.float32).reshape(B, 1)
    w2t = W2.T
    w2p = jnp.concatenate([w2t[:48], jnp.zeros((4, D), jnp.float32), w2t[48:]], axis=0)
    return _mlp(pooled, lenf, w2p, b2.reshape(1, D), W1.T, b1.reshape(1, D))


# speedup vs baseline: 11.9414x; 11.9414x over previous
import functools

import jax
import jax.numpy as jnp
from jax import lax
from jax.experimental import pallas as pl
from jax.experimental.pallas import tpu as pltpu
from jax.experimental.pallas import tpu_sc as plsc

B = 4096
L = 200
D = 60
DP = 64
NC = 2
NS = 16
NW = NC * NS
RPW = B // NW
HALF = L // 2


def _pool_body(x3_hbm, tbl_hbm, out_hbm, idx_v, buf0, buf1, stage, sem0, sem1):
    wid = lax.axis_index("s") * NC + lax.axis_index("c")
    pltpu.sync_copy(x3_hbm.at[wid], idx_v)

    def fire(row, buf, sem):
        pltpu.async_copy(tbl_hbm.at[idx_v.at[2 * row]], buf.at[pl.ds(0, HALF)], sem)
        pltpu.async_copy(tbl_hbm.at[idx_v.at[2 * row + 1]], buf.at[pl.ds(HALF, HALF)], sem)

    def drain(row, buf, sem):
        pltpu.make_async_copy(tbl_hbm.at[idx_v.at[2 * row]], buf.at[pl.ds(0, HALF)], sem).wait()
        pltpu.make_async_copy(tbl_hbm.at[idx_v.at[2 * row + 1]], buf.at[pl.ds(HALF, HALF)], sem).wait()

    def accum(buf, row):
        zero = jnp.zeros((16,), jnp.float32)

        def body(i, accs):
            a0, a1, a2, a3 = accs
            for u in range(2):
                r = 2 * i + u
                a0 = a0 + buf[r, pl.ds(0, 16)]
                a1 = a1 + buf[r, pl.ds(16, 16)]
                a2 = a2 + buf[r, pl.ds(32, 16)]
                a3 = a3 + buf[r, pl.ds(48, 16)]
            return a0, a1, a2, a3

        a0, a1, a2, a3 = lax.fori_loop(0, L // 2, body, (zero, zero, zero, zero))
        stage[row, pl.ds(0, 16)] = a0
        stage[row, pl.ds(16, 16)] = a1
        stage[row, pl.ds(32, 16)] = a2
        stage[row, pl.ds(48, 16)] = a3

    fire(0, buf0, sem0)
    fire(1, buf1, sem1)

    def outer(g, carry):
        b0 = 2 * g
        drain(b0, buf0, sem0)
        accum(buf0, b0)

        @pl.when(g < RPW // 2 - 1)
        def _():
            fire(b0 + 2, buf0, sem0)

        drain(b0 + 1, buf1, sem1)
        accum(buf1, b0 + 1)

        @pl.when(g < RPW // 2 - 1)
        def _():
            fire(b0 + 3, buf1, sem1)

        return carry

    lax.fori_loop(0, RPW // 2, outer, 0)
    pltpu.sync_copy(stage, out_hbm.at[wid])


_pool = functools.partial(
    pl.kernel,
    out_type=jax.ShapeDtypeStruct((NW, RPW, DP), jnp.float32),
    mesh=plsc.VectorSubcoreMesh(core_axis_name="c", subcore_axis_name="s"),
    compiler_params=pltpu.CompilerParams(use_tc_tiling_on_sc=False),
    scratch_types=[
        pltpu.VMEM((2 * RPW, HALF), jnp.int32),
        pltpu.VMEM((L, DP), jnp.float32),
        pltpu.VMEM((L, DP), jnp.float32),
        pltpu.VMEM((RPW, DP), jnp.float32),
        pltpu.SemaphoreType.DMA,
        pltpu.SemaphoreType.DMA,
    ],
)(_pool_body)


def _mlp_body(p_ref, il_ref, w2_ref, b2_ref, w1_ref, b1_ref, o_ref):
    x = p_ref[...] / il_ref[...]
    h = jnp.dot(x, w2_ref[...], preferred_element_type=jnp.float32) + b2_ref[...]
    h = jnp.maximum(h, 0.0)
    o_ref[...] = jnp.dot(h, w1_ref[...], preferred_element_type=jnp.float32) + b1_ref[...]


BT = 512


def _mlp(pooled, lenf, w2p, b2r, w1t, b1r):
    return pl.pallas_call(
        _mlp_body,
        grid=(B // BT,),
        in_specs=[
            pl.BlockSpec((BT, DP), lambda i: (i, 0)),
            pl.BlockSpec((BT, 1), lambda i: (i, 0)),
            pl.BlockSpec((DP, D), lambda i: (0, 0)),
            pl.BlockSpec((1, D), lambda i: (0, 0)),
            pl.BlockSpec((D, D), lambda i: (0, 0)),
            pl.BlockSpec((1, D), lambda i: (0, 0)),
        ],
        out_specs=pl.BlockSpec((BT, D), lambda i: (i, 0)),
        out_shape=jax.ShapeDtypeStruct((B, D), jnp.float32),
    )(pooled, lenf, w2p, b2r, w1t, b1r)


def kernel(x, lengths, table, W2, b2, W1, b1):
    x3 = x.astype(jnp.int32).reshape(NW, 2 * RPW, HALF)
    tbl = jnp.pad(table, ((0, 0), (0, DP - D)))
    pooled = _pool(x3, tbl).reshape(B, DP)
    lenf = lengths.astype(jnp.float32).reshape(B, 1)
    w2p = jnp.concatenate([W2.T, jnp.zeros((DP - D, D), jnp.float32)], axis=0)
    return _mlp(pooled, lenf, w2p, b2.reshape(1, D), W1.T, b1.reshape(1, D))


# speedup vs baseline: 12.0577x; 1.0097x over previous
import functools

import jax
import jax.numpy as jnp
from jax import lax
from jax.experimental import pallas as pl
from jax.experimental.pallas import tpu as pltpu
from jax.experimental.pallas import tpu_sc as plsc

B = 4096
L = 200
D = 60
DP = 64
NC = 2
NS = 16
NW = NC * NS
RPW = B // NW
GA = 104
GB = L - GA


def _pool_body(x3_hbm, tbl_hbm, out_hbm, idx_v, buf0, buf1, stage, sem0, sem1):
    wid = lax.axis_index("s") * NC + lax.axis_index("c")
    pltpu.sync_copy(x3_hbm.at[wid], idx_v)

    def fire(row, buf, sem):
        pltpu.async_copy(tbl_hbm.at[idx_v.at[row, pl.ds(0, GA)]], buf.at[pl.ds(0, GA)], sem)
        pltpu.async_copy(tbl_hbm.at[idx_v.at[row, pl.ds(GA, GB)]], buf.at[pl.ds(GA, GB)], sem)

    def drain(row, buf, sem):
        pltpu.make_async_copy(tbl_hbm.at[idx_v.at[row, pl.ds(0, GA)]], buf.at[pl.ds(0, GA)], sem).wait()
        pltpu.make_async_copy(tbl_hbm.at[idx_v.at[row, pl.ds(GA, GB)]], buf.at[pl.ds(GA, GB)], sem).wait()

    def accum(buf, row):
        zero = jnp.zeros((16,), jnp.float32)

        def body(i, accs):
            a0, a1, a2, a3 = accs
            for u in range(2):
                r = 2 * i + u
                a0 = a0 + buf[r, pl.ds(0, 16)]
                a1 = a1 + buf[r, pl.ds(16, 16)]
                a2 = a2 + buf[r, pl.ds(32, 16)]
                a3 = a3 + buf[r, pl.ds(48, 16)]
            return a0, a1, a2, a3

        a0, a1, a2, a3 = lax.fori_loop(0, L // 2, body, (zero, zero, zero, zero))
        stage[row, pl.ds(0, 16)] = a0
        stage[row, pl.ds(16, 16)] = a1
        stage[row, pl.ds(32, 16)] = a2
        stage[row, pl.ds(48, 16)] = a3

    fire(0, buf0, sem0)
    fire(1, buf1, sem1)

    def outer(g, carry):
        b0 = 2 * g
        drain(b0, buf0, sem0)
        accum(buf0, b0)

        @pl.when(g < RPW // 2 - 1)
        def _():
            fire(b0 + 2, buf0, sem0)

        drain(b0 + 1, buf1, sem1)
        accum(buf1, b0 + 1)

        @pl.when(g < RPW // 2 - 1)
        def _():
            fire(b0 + 3, buf1, sem1)

        return carry

    lax.fori_loop(0, RPW // 2, outer, 0)
    pltpu.sync_copy(stage, out_hbm.at[wid])


_pool = functools.partial(
    pl.kernel,
    out_type=jax.ShapeDtypeStruct((NW, RPW, DP), jnp.float32),
    mesh=plsc.VectorSubcoreMesh(core_axis_name="c", subcore_axis_name="s"),
    compiler_params=pltpu.CompilerParams(use_tc_tiling_on_sc=False),
    scratch_types=[
        pltpu.VMEM((RPW, L), jnp.int32),
        pltpu.VMEM((L, DP), jnp.float32),
        pltpu.VMEM((L, DP), jnp.float32),
        pltpu.VMEM((RPW, DP), jnp.float32),
        pltpu.SemaphoreType.DMA,
        pltpu.SemaphoreType.DMA,
    ],
)(_pool_body)


def _mlp_body(p_ref, il_ref, w2_ref, b2_ref, w1_ref, b1_ref, o_ref):
    x = p_ref[...] / il_ref[...]
    h = jnp.dot(x, w2_ref[...], preferred_element_type=jnp.float32) + b2_ref[...]
    h = jnp.maximum(h, 0.0)
    o_ref[...] = jnp.dot(h, w1_ref[...], preferred_element_type=jnp.float32) + b1_ref[...]


BT = 512


def _mlp(pooled, lenf, w2p, b2r, w1t, b1r):
    return pl.pallas_call(
        _mlp_body,
        grid=(B // BT,),
        in_specs=[
            pl.BlockSpec((BT, DP), lambda i: (i, 0)),
            pl.BlockSpec((BT, 1), lambda i: (i, 0)),
            pl.BlockSpec((DP, D), lambda i: (0, 0)),
            pl.BlockSpec((1, D), lambda i: (0, 0)),
            pl.BlockSpec((D, D), lambda i: (0, 0)),
            pl.BlockSpec((1, D), lambda i: (0, 0)),
        ],
        out_specs=pl.BlockSpec((BT, D), lambda i: (i, 0)),
        out_shape=jax.ShapeDtypeStruct((B, D), jnp.float32),
    )(pooled, lenf, w2p, b2r, w1t, b1r)


def kernel(x, lengths, table, W2, b2, W1, b1):
    x3 = x.astype(jnp.int32).reshape(NW, RPW, L)
    tbl = jnp.pad(table, ((0, 0), (0, DP - D)))
    pooled = _pool(x3, tbl).reshape(B, DP)
    lenf = lengths.astype(jnp.float32).reshape(B, 1)
    w2p = jnp.concatenate([W2.T, jnp.zeros((DP - D, D), jnp.float32)], axis=0)
    return _mlp(pooled, lenf, w2p, b2.reshape(1, D), W1.T, b1.reshape(1, D))


# speedup vs baseline: 12.0858x; 1.0023x over previous
import functools

import jax
import jax.numpy as jnp
from jax import lax
from jax.experimental import pallas as pl
from jax.experimental.pallas import tpu as pltpu
from jax.experimental.pallas import tpu_sc as plsc

B = 4096
L = 200
D = 60
DP = 64
NC = 2
NS = 16
NW = NC * NS
RPW = B // NW
QOFF = (0, 56, 112, 168)
QCNT = (56, 56, 56, 32)
QROWS = 56


def _pool_body(x3_hbm, tbl_hbm, out_hbm, idx_v, buf0, buf1, stage, sem0, sem1):
    wid = lax.axis_index("s") * NC + lax.axis_index("c")
    pltpu.sync_copy(x3_hbm.at[wid], idx_v)

    zero = jnp.zeros((16,), jnp.float32)

    def zero_buf(buf):
        def zrow(r, carry):
            buf[r, pl.ds(0, 16)] = zero
            buf[r, pl.ds(16, 16)] = zero
            buf[r, pl.ds(32, 16)] = zero
            buf[r, pl.ds(48, 16)] = zero
            return carry

        lax.fori_loop(0, QROWS, zrow, 0)

    def fire(row, buf, sem):
        for off, cnt in zip(QOFF, QCNT):
            pltpu.async_copy(
                tbl_hbm.at[idx_v.at[row, pl.ds(off, cnt)]],
                buf.at[pl.ds(0, cnt)], sem, add=True)

    def drain(row, buf, sem):
        for off, cnt in zip(QOFF, QCNT):
            pltpu.make_async_copy(
                tbl_hbm.at[idx_v.at[row, pl.ds(off, cnt)]],
                buf.at[pl.ds(0, cnt)], sem).wait()

    def accum(buf, row):
        def body(i, accs):
            a0, a1, a2, a3 = accs
            a0 = a0 + buf[i, pl.ds(0, 16)]
            a1 = a1 + buf[i, pl.ds(16, 16)]
            a2 = a2 + buf[i, pl.ds(32, 16)]
            a3 = a3 + buf[i, pl.ds(48, 16)]
            buf[i, pl.ds(0, 16)] = zero
            buf[i, pl.ds(16, 16)] = zero
            buf[i, pl.ds(32, 16)] = zero
            buf[i, pl.ds(48, 16)] = zero
            return a0, a1, a2, a3

        a0, a1, a2, a3 = lax.fori_loop(0, QROWS, body, (zero, zero, zero, zero))
        stage[row, pl.ds(0, 16)] = a0
        stage[row, pl.ds(16, 16)] = a1
        stage[row, pl.ds(32, 16)] = a2
        stage[row, pl.ds(48, 16)] = a3

    zero_buf(buf0)
    zero_buf(buf1)
    fire(0, buf0, sem0)
    fire(1, buf1, sem1)

    def outer(g, carry):
        b0 = 2 * g
        drain(b0, buf0, sem0)
        accum(buf0, b0)

        @pl.when(g < RPW // 2 - 1)
        def _():
            fire(b0 + 2, buf0, sem0)

        drain(b0 + 1, buf1, sem1)
        accum(buf1, b0 + 1)

        @pl.when(g < RPW // 2 - 1)
        def _():
            fire(b0 + 3, buf1, sem1)

        return carry

    lax.fori_loop(0, RPW // 2, outer, 0)
    pltpu.sync_copy(stage, out_hbm.at[wid])


_pool = functools.partial(
    pl.kernel,
    out_type=jax.ShapeDtypeStruct((NW, RPW, DP), jnp.float32),
    mesh=plsc.VectorSubcoreMesh(core_axis_name="c", subcore_axis_name="s"),
    compiler_params=pltpu.CompilerParams(use_tc_tiling_on_sc=False),
    scratch_types=[
        pltpu.VMEM((RPW, L), jnp.int32),
        pltpu.VMEM((QROWS, DP), jnp.float32),
        pltpu.VMEM((QROWS, DP), jnp.float32),
        pltpu.VMEM((RPW, DP), jnp.float32),
        pltpu.SemaphoreType.DMA,
        pltpu.SemaphoreType.DMA,
    ],
)(_pool_body)


def _mlp_body(p_ref, il_ref, w2_ref, b2_ref, w1_ref, b1_ref, o_ref):
    x = p_ref[...] / il_ref[...]
    h = jnp.dot(x, w2_ref[...], preferred_element_type=jnp.float32) + b2_ref[...]
    h = jnp.maximum(h, 0.0)
    o_ref[...] = jnp.dot(h, w1_ref[...], preferred_element_type=jnp.float32) + b1_ref[...]


BT = 512


def _mlp(pooled, lenf, w2p, b2r, w1t, b1r):
    return pl.pallas_call(
        _mlp_body,
        grid=(B // BT,),
        in_specs=[
            pl.BlockSpec((BT, DP), lambda i: (i, 0)),
            pl.BlockSpec((BT, 1), lambda i: (i, 0)),
            pl.BlockSpec((DP, D), lambda i: (0, 0)),
            pl.BlockSpec((1, D), lambda i: (0, 0)),
            pl.BlockSpec((D, D), lambda i: (0, 0)),
            pl.BlockSpec((1, D), lambda i: (0, 0)),
        ],
        out_specs=pl.BlockSpec((BT, D), lambda i: (i, 0)),
        out_shape=jax.ShapeDtypeStruct((B, D), jnp.float32),
    )(pooled, lenf, w2p, b2r, w1t, b1r)


def kernel(x, lengths, table, W2, b2, W1, b1):
    x3 = x.astype(jnp.int32).reshape(NW, RPW, L)
    tbl = jnp.pad(table, ((0, 0), (0, DP - D)))
    pooled = _pool(x3, tbl).reshape(B, DP)
    lenf = lengths.astype(jnp.float32).reshape(B, 1)
    w2p = jnp.concatenate([W2.T, jnp.zeros((DP - D, D), jnp.float32)], axis=0)
    return _mlp(pooled, lenf, w2p, b2.reshape(1, D), W1.T, b1.reshape(1, D))


# speedup vs baseline: 14.5200x; 1.2014x over previous
import functools

import jax
import jax.numpy as jnp
from jax import lax
from jax.experimental import pallas as pl
from jax.experimental.pallas import tpu as pltpu
from jax.experimental.pallas import tpu_sc as plsc

B = 4096
L = 200
D = 60
DP = 64
NC = 2
NS = 16
NW = NC * NS
RPW = B // NW
QOFF = (0, 56, 112, 168)
QCNT = (56, 56, 56, 32)
QROWS = 56


def _pool_body(x3_hbm, tbl_hbm, out_hbm, idx_v, buf0, buf1, stage, sem0, sem1):
    wid = lax.axis_index("s") * NC + lax.axis_index("c")
    pltpu.sync_copy(x3_hbm.at[wid], idx_v)

    zero = jnp.zeros((16,), jnp.float32)

    def zero_buf(buf):
        def zrow(r, carry):
            buf[r, pl.ds(0, 16)] = zero
            buf[r, pl.ds(16, 16)] = zero
            buf[r, pl.ds(32, 16)] = zero
            buf[r, pl.ds(48, 16)] = zero
            return carry

        lax.fori_loop(0, QROWS, zrow, 0)

    def fire(row, buf, sem):
        for off, cnt in zip(QOFF, QCNT):
            pltpu.async_copy(
                tbl_hbm.at[idx_v.at[row, pl.ds(off, cnt)]],
                buf.at[pl.ds(0, cnt)], sem, add=True)

    def drain(row, buf, sem):
        for off, cnt in zip(QOFF, QCNT):
            pltpu.make_async_copy(
                tbl_hbm.at[idx_v.at[row, pl.ds(off, cnt)]],
                buf.at[pl.ds(0, cnt)], sem).wait()

    def accum(buf, row):
        def body(i, accs):
            a0, a1, a2, a3 = accs
            a0 = a0 + buf[i, pl.ds(0, 16)]
            a1 = a1 + buf[i, pl.ds(16, 16)]
            a2 = a2 + buf[i, pl.ds(32, 16)]
            a3 = a3 + buf[i, pl.ds(48, 16)]
            buf[i, pl.ds(0, 16)] = zero
            buf[i, pl.ds(16, 16)] = zero
            buf[i, pl.ds(32, 16)] = zero
            buf[i, pl.ds(48, 16)] = zero
            return a0, a1, a2, a3

        a0, a1, a2, a3 = lax.fori_loop(0, QROWS, body, (zero, zero, zero, zero))
        stage[row, pl.ds(0, 16)] = a0
        stage[row, pl.ds(16, 16)] = a1
        stage[row, pl.ds(32, 16)] = a2
        stage[row, pl.ds(48, 16)] = a3

    zero_buf(buf0)
    zero_buf(buf1)
    fire(0, buf0, sem0)
    fire(1, buf1, sem1)

    def outer(g, carry):
        b0 = 2 * g
        drain(b0, buf0, sem0)
        accum(buf0, b0)

        @pl.when(g < RPW // 2 - 1)
        def _():
            fire(b0 + 2, buf0, sem0)

        drain(b0 + 1, buf1, sem1)
        accum(buf1, b0 + 1)

        @pl.when(g < RPW // 2 - 1)
        def _():
            fire(b0 + 3, buf1, sem1)

        return carry

    lax.fori_loop(0, RPW // 2, outer, 0)
    pltpu.sync_copy(stage, out_hbm.at[wid])


_pool = functools.partial(
    pl.kernel,
    out_type=jax.ShapeDtypeStruct((NW, RPW, DP), jnp.float32),
    mesh=plsc.VectorSubcoreMesh(core_axis_name="c", subcore_axis_name="s"),
    compiler_params=pltpu.CompilerParams(use_tc_tiling_on_sc=False),
    scratch_types=[
        pltpu.VMEM((RPW, L), jnp.int32),
        pltpu.VMEM((QROWS, DP), jnp.float32),
        pltpu.VMEM((QROWS, DP), jnp.float32),
        pltpu.VMEM((RPW, DP), jnp.float32),
        pltpu.SemaphoreType.DMA,
        pltpu.SemaphoreType.DMA,
    ],
)(_pool_body)


def _mlp_body(p_ref, il_ref, w2_ref, b2_ref, w1_ref, b1_ref, o_ref):
    x = p_ref[...] / il_ref[...]
    h = jnp.dot(x, w2_ref[...], preferred_element_type=jnp.float32) + b2_ref[...]
    h = jnp.maximum(h, 0.0)
    o_ref[...] = jnp.dot(h, w1_ref[...], preferred_element_type=jnp.float32) + b1_ref[...]


BT = 512


def _mlp(pooled, lenf, w2p, b2r, w1t, b1r):
    return pl.pallas_call(
        _mlp_body,
        grid=(B // BT,),
        in_specs=[
            pl.BlockSpec((BT, DP), lambda i: (i, 0)),
            pl.BlockSpec((BT, 1), lambda i: (i, 0)),
            pl.BlockSpec((DP, D), lambda i: (0, 0)),
            pl.BlockSpec((1, D), lambda i: (0, 0)),
            pl.BlockSpec((D, D), lambda i: (0, 0)),
            pl.BlockSpec((1, D), lambda i: (0, 0)),
        ],
        out_specs=pl.BlockSpec((BT, D), lambda i: (i, 0)),
        out_shape=jax.ShapeDtypeStruct((B, D), jnp.float32),
    )(pooled, lenf, w2p, b2r, w1t, b1r)


def kernel(x, lengths, table, W2, b2, W1, b1):
    x3 = (x.astype(jnp.int32) * 2).reshape(NW, RPW, L)
    tbl = jnp.pad(table, ((0, 0), (0, 128 - D))).reshape(-1, DP)
    pooled = _pool(x3, tbl).reshape(B, DP)
    lenf = lengths.astype(jnp.float32).reshape(B, 1)
    w2p = jnp.concatenate([W2.T, jnp.zeros((DP - D, D), jnp.float32)], axis=0)
    return _mlp(pooled, lenf, w2p, b2.reshape(1, D), W1.T, b1.reshape(1, D))
